# Initial kernel scaffold; baseline (speedup 1.0000x reference)
import jax
import jax.numpy as jnp
from jax.experimental import pallas as pl


def kernel(x, target, nwords):
    raise NotImplementedError("write your pallas kernel here")


# speedup vs baseline: 1.8377x; 1.8377x over previous
import functools
import math

import jax
import jax.numpy as jnp
import numpy as np
from jax.experimental import pallas as pl
from jax.experimental.pallas import tpu as pltpu

_SIZE = 100000
_SMOOTHING = 0.1
_PAD_ID = 3

_EPS = np.float32(_SMOOTHING / (_SIZE - 2))
_ROW_CONST = float(
    (_SIZE - 2) * (_EPS * np.log(_EPS))
    + np.float32(1.0 - _SMOOTHING) * np.log(np.float32(1.0 - _SMOOTHING))
)

_BC = 2048


def _kl_kernel(t_ref, x_ref, out_ref):
    j = pl.program_id(0)
    nblk = pl.num_programs(0)

    t = t_ref[:, :]
    x = x_ref[:, :]
    n, bc = x.shape

    c0 = j * bc
    col = jax.lax.broadcasted_iota(jnp.int32, (n, bc), 1) + c0
    row_ok = t != _PAD_ID

    w = jnp.where(col == t, _EPS - (1.0 - _SMOOTHING), -_EPS)
    w = jnp.where(col == _PAD_ID, jnp.float32(0.0), w)
    w = jnp.where(row_ok, w, jnp.float32(0.0))

    x = jnp.where(col < _SIZE, x, jnp.float32(0.0))
    contrib = jnp.sum(x * w)

    @pl.when(j == 0)
    def _init():
        out_ref[:, :] = jnp.zeros((1, 1), jnp.float32)

    out_ref[:, :] += contrib.reshape(1, 1)

    @pl.when(j == nblk - 1)
    def _fini():
        count = jnp.sum(row_ok.astype(jnp.float32))
        out_ref[:, :] += (jnp.float32(_ROW_CONST) * count).reshape(1, 1)


@jax.jit
def _run(x, t):
    n = x.shape[0]
    nblk = pl.cdiv(_SIZE, _BC)
    out = pl.pallas_call(
        _kl_kernel,
        grid=(nblk,),
        in_specs=[
            pl.BlockSpec((n, 1), lambda j: (0, 0)),
            pl.BlockSpec((n, _BC), lambda j: (0, j)),
        ],
        out_specs=pl.BlockSpec((1, 1), lambda j: (0, 0)),
        out_shape=jax.ShapeDtypeStruct((1, 1), jnp.float32),
    )(t, x)
    return out[0, 0]


def kernel(x, target, nwords):
    x2 = x.reshape(-1, _SIZE)
    t = target.reshape(-1).astype(jnp.int32)[:, None]
    kl = _run(x2, t)
    return kl / nwords
